# baseline (device time: 514574 ns/iter reference)
import jax
import jax.numpy as jnp
from jax import lax
from jax.experimental import pallas as pl
from jax.experimental.pallas import tpu as pltpu

N_EXPERTS = 4
EXPERTS_PER_SHARD = 2
CAP = 320


def kernel(x, assign, W1, W2):
    tokens, d_model = x.shape
    my_x = lax.axis_index("x")

    oh = (assign[:, None] == jnp.arange(N_EXPERTS, dtype=assign.dtype)[None, :]).astype(jnp.int32)
    pos = jnp.take_along_axis(jnp.cumsum(oh, axis=0) - 1, assign[:, None], axis=1)[:, 0]
    tok_ids = jnp.arange(tokens, dtype=jnp.int32)
    idx = jnp.zeros((N_EXPERTS, CAP), jnp.int32).at[assign, pos].set(tok_ids, mode="drop")
    valid = jnp.zeros((N_EXPERTS, CAP), x.dtype).at[assign, pos].set(1.0, mode="drop")

    xg = jnp.take(x, idx.reshape(-1), axis=0).reshape(N_EXPERTS, CAP, d_model)

    loc0 = EXPERTS_PER_SHARD * my_x
    out0 = EXPERTS_PER_SHARD * (1 - my_x)
    xg_loc = lax.dynamic_slice(xg, (loc0, 0, 0), (EXPERTS_PER_SHARD, CAP, d_model))
    xg_out = lax.dynamic_slice(xg, (out0, 0, 0), (EXPERTS_PER_SHARD, CAP, d_model))

    def body(xl_ref, xo_ref, w1_ref, w2_ref, resl_ref, resb_ref,
             xin, resout, send_sems, recv_sems):
        mx = lax.axis_index("x")
        my = lax.axis_index("y")
        mz = lax.axis_index("z")
        peer = (1 - mx, my, mz)

        barrier_sem = pltpu.get_barrier_semaphore()
        pl.semaphore_signal(barrier_sem, inc=1, device_id=peer,
                            device_id_type=pl.DeviceIdType.MESH)
        pl.semaphore_wait(barrier_sem, 1)

        rdma_x = []
        for k in range(EXPERTS_PER_SHARD):
            r = pltpu.make_async_remote_copy(
                src_ref=xo_ref.at[k], dst_ref=xin.at[k],
                send_sem=send_sems.at[k], recv_sem=recv_sems.at[k],
                device_id=peer, device_id_type=pl.DeviceIdType.MESH)
            r.start()
            rdma_x.append(r)

        for k in range(EXPERTS_PER_SHARD):
            h = jnp.maximum(jnp.dot(xl_ref[k], w1_ref[k], preferred_element_type=jnp.float32), 0.0)
            resl_ref[k] = jnp.dot(h, w2_ref[k], preferred_element_type=jnp.float32)

        rdma_r = []
        for k in range(EXPERTS_PER_SHARD):
            rdma_x[k].wait_recv()
            h = jnp.maximum(jnp.dot(xin[k], w1_ref[k], preferred_element_type=jnp.float32), 0.0)
            resout[k] = jnp.dot(h, w2_ref[k], preferred_element_type=jnp.float32)
            r = pltpu.make_async_remote_copy(
                src_ref=resout.at[k], dst_ref=resb_ref.at[k],
                send_sem=send_sems.at[EXPERTS_PER_SHARD + k],
                recv_sem=recv_sems.at[EXPERTS_PER_SHARD + k],
                device_id=peer, device_id_type=pl.DeviceIdType.MESH)
            r.start()
            rdma_r.append(r)

        for r in rdma_r:
            r.wait()
        for r in rdma_x:
            r.wait_send()

    resl, resb = pl.pallas_call(
        body,
        out_shape=[
            jax.ShapeDtypeStruct((EXPERTS_PER_SHARD, CAP, d_model), x.dtype),
            jax.ShapeDtypeStruct((EXPERTS_PER_SHARD, CAP, d_model), x.dtype),
        ],
        in_specs=[pl.BlockSpec(memory_space=pltpu.VMEM)] * 4,
        out_specs=[pl.BlockSpec(memory_space=pltpu.VMEM)] * 2,
        scratch_shapes=[
            pltpu.VMEM((EXPERTS_PER_SHARD, CAP, d_model), x.dtype),
            pltpu.VMEM((EXPERTS_PER_SHARD, CAP, d_model), x.dtype),
            pltpu.SemaphoreType.DMA((2 * EXPERTS_PER_SHARD,)),
            pltpu.SemaphoreType.DMA((2 * EXPERTS_PER_SHARD,)),
        ],
        compiler_params=pltpu.CompilerParams(
            collective_id=0,
            vmem_limit_bytes=100 * 1024 * 1024,
        ),
    )(xg_loc, xg_out, W1, W2)

    idx_loc = lax.dynamic_slice(idx, (loc0, 0), (EXPERTS_PER_SHARD, CAP))
    idx_out = lax.dynamic_slice(idx, (out0, 0), (EXPERTS_PER_SHARD, CAP))
    val_loc = lax.dynamic_slice(valid, (loc0, 0), (EXPERTS_PER_SHARD, CAP))
    val_out = lax.dynamic_slice(valid, (out0, 0), (EXPERTS_PER_SHARD, CAP))

    vals = jnp.concatenate([
        (resl * val_loc[:, :, None]).reshape(-1, d_model),
        (resb * val_out[:, :, None]).reshape(-1, d_model),
    ])
    idxs = jnp.concatenate([idx_loc.reshape(-1), idx_out.reshape(-1)])
    return jnp.zeros_like(x).at[idxs].add(vals)


# device time: 94853 ns/iter; 5.4250x vs baseline; 5.4250x over previous
import jax
import jax.numpy as jnp
from jax import lax
from jax.experimental import pallas as pl
from jax.experimental.pallas import tpu as pltpu

N_EXPERTS = 4
EXPERTS_PER_SHARD = 2
CAP = 320


def kernel(x, assign, W1, W2):
    tokens, d_model = x.shape
    my_x = lax.axis_index("x")

    oh = (assign[:, None] == jnp.arange(N_EXPERTS, dtype=assign.dtype)[None, :])
    pos = jnp.cumsum(oh.astype(jnp.int32), axis=0) - 1
    sel = oh.T[:, None, :] & (pos.T[:, None, :] == jnp.arange(CAP, dtype=jnp.int32)[None, :, None])
    P = sel.astype(x.dtype)
    PT = jnp.swapaxes(P, 1, 2)

    loc0 = EXPERTS_PER_SHARD * my_x
    out0 = EXPERTS_PER_SHARD * (1 - my_x)
    p_loc = lax.dynamic_slice(P, (loc0, 0, 0), (EXPERTS_PER_SHARD, CAP, tokens))
    p_out = lax.dynamic_slice(P, (out0, 0, 0), (EXPERTS_PER_SHARD, CAP, tokens))
    pt_loc = lax.dynamic_slice(PT, (loc0, 0, 0), (EXPERTS_PER_SHARD, tokens, CAP))
    pt_out = lax.dynamic_slice(PT, (out0, 0, 0), (EXPERTS_PER_SHARD, tokens, CAP))

    def body(x_ref, pl_ref, po_ref, ptl_ref, pto_ref, w1_ref, w2_ref,
             out_ref, xin, resout, resb, send_sems, recv_sems):
        mx = lax.axis_index("x")
        my = lax.axis_index("y")
        mz = lax.axis_index("z")
        peer = (1 - mx, my, mz)

        barrier_sem = pltpu.get_barrier_semaphore()
        pl.semaphore_signal(barrier_sem, inc=1, device_id=peer,
                            device_id_type=pl.DeviceIdType.MESH)
        pl.semaphore_wait(barrier_sem, 1)

        xv = x_ref[...]

        rdma_x = []
        for k in range(EXPERTS_PER_SHARD):
            resout[k] = jnp.dot(po_ref[k], xv, preferred_element_type=jnp.float32)
            r = pltpu.make_async_remote_copy(
                src_ref=resout.at[k], dst_ref=xin.at[k],
                send_sem=send_sems.at[k], recv_sem=recv_sems.at[k],
                device_id=peer, device_id_type=pl.DeviceIdType.MESH)
            r.start()
            rdma_x.append(r)

        acc = jnp.zeros((tokens, d_model), jnp.float32)
        for k in range(EXPERTS_PER_SHARD):
            xg = jnp.dot(pl_ref[k], xv, preferred_element_type=jnp.float32)
            h = jnp.maximum(jnp.dot(xg, w1_ref[k], preferred_element_type=jnp.float32), 0.0)
            res = jnp.dot(h, w2_ref[k], preferred_element_type=jnp.float32)
            acc = acc + jnp.dot(ptl_ref[k], res, preferred_element_type=jnp.float32)
        out_ref[...] = acc

        rdma_r = []
        for k in range(EXPERTS_PER_SHARD):
            rdma_x[k].wait_send()
            rdma_x[k].wait_recv()
            h = jnp.maximum(jnp.dot(xin[k], w1_ref[k], preferred_element_type=jnp.float32), 0.0)
            resout[k] = jnp.dot(h, w2_ref[k], preferred_element_type=jnp.float32)
            r = pltpu.make_async_remote_copy(
                src_ref=resout.at[k], dst_ref=resb.at[k],
                send_sem=send_sems.at[EXPERTS_PER_SHARD + k],
                recv_sem=recv_sems.at[EXPERTS_PER_SHARD + k],
                device_id=peer, device_id_type=pl.DeviceIdType.MESH)
            r.start()
            rdma_r.append(r)

        for k in range(EXPERTS_PER_SHARD):
            rdma_r[k].wait_recv()
            out_ref[...] = out_ref[...] + jnp.dot(
                pto_ref[k], resb[k], preferred_element_type=jnp.float32)
        for r in rdma_r:
            r.wait_send()

    return pl.pallas_call(
        body,
        out_shape=jax.ShapeDtypeStruct((tokens, d_model), x.dtype),
        in_specs=[pl.BlockSpec(memory_space=pltpu.VMEM)] * 7,
        out_specs=pl.BlockSpec(memory_space=pltpu.VMEM),
        scratch_shapes=[
            pltpu.VMEM((EXPERTS_PER_SHARD, CAP, d_model), x.dtype),
            pltpu.VMEM((EXPERTS_PER_SHARD, CAP, d_model), x.dtype),
            pltpu.VMEM((EXPERTS_PER_SHARD, CAP, d_model), x.dtype),
            pltpu.SemaphoreType.DMA((2 * EXPERTS_PER_SHARD,)),
            pltpu.SemaphoreType.DMA((2 * EXPERTS_PER_SHARD,)),
        ],
        compiler_params=pltpu.CompilerParams(
            collective_id=0,
            vmem_limit_bytes=100 * 1024 * 1024,
        ),
    )(x, p_loc, p_out, pt_loc, pt_out, W1, W2)


# device time: 94820 ns/iter; 5.4269x vs baseline; 1.0003x over previous
import jax
import jax.numpy as jnp
from jax import lax
from jax.experimental import pallas as pl
from jax.experimental.pallas import tpu as pltpu

N_EXPERTS = 4
EXPERTS_PER_SHARD = 2
CAP = 320


def kernel(x, assign, W1, W2):
    tokens, d_model = x.shape
    my_x = lax.axis_index("x")

    oh = (assign[:, None] == jnp.arange(N_EXPERTS, dtype=assign.dtype)[None, :])
    pos = jnp.cumsum(oh.astype(jnp.int32), axis=0) - 1
    sel = oh.T[:, None, :] & (pos.T[:, None, :] == jnp.arange(CAP, dtype=jnp.int32)[None, :, None])
    P = sel.astype(x.dtype)
    PT = jnp.swapaxes(P, 1, 2)

    loc0 = EXPERTS_PER_SHARD * my_x
    out0 = EXPERTS_PER_SHARD * (1 - my_x)
    p_loc = lax.dynamic_slice(P, (loc0, 0, 0), (EXPERTS_PER_SHARD, CAP, tokens))
    p_out = lax.dynamic_slice(P, (out0, 0, 0), (EXPERTS_PER_SHARD, CAP, tokens))
    pt_loc = lax.dynamic_slice(PT, (loc0, 0, 0), (EXPERTS_PER_SHARD, tokens, CAP))
    pt_out = lax.dynamic_slice(PT, (out0, 0, 0), (EXPERTS_PER_SHARD, tokens, CAP))

    def body(x_ref, pl_ref, po_ref, ptl_ref, pto_ref, w1_ref, w2_ref,
             out_ref, xin, resout, resb, send_sems, recv_sems):
        mx = lax.axis_index("x")
        my = lax.axis_index("y")
        mz = lax.axis_index("z")
        peer = (1 - mx, my, mz)

        barrier_sem = pltpu.get_barrier_semaphore()
        pl.semaphore_signal(barrier_sem, inc=1, device_id=peer,
                            device_id_type=pl.DeviceIdType.MESH)
        pl.semaphore_wait(barrier_sem, 1)

        xv = x_ref[...]

        rdma_x = []
        for k in range(EXPERTS_PER_SHARD):
            resout[k] = jnp.dot(po_ref[k], xv, preferred_element_type=jnp.float32, precision=lax.Precision.DEFAULT)
            r = pltpu.make_async_remote_copy(
                src_ref=resout.at[k], dst_ref=xin.at[k],
                send_sem=send_sems.at[k], recv_sem=recv_sems.at[k],
                device_id=peer, device_id_type=pl.DeviceIdType.MESH)
            r.start()
            rdma_x.append(r)

        acc = jnp.zeros((tokens, d_model), jnp.float32)
        for k in range(EXPERTS_PER_SHARD):
            xg = jnp.dot(pl_ref[k], xv, preferred_element_type=jnp.float32, precision=lax.Precision.DEFAULT)
            h = jnp.maximum(jnp.dot(xg, w1_ref[k], preferred_element_type=jnp.float32, precision=lax.Precision.DEFAULT), 0.0)
            res = jnp.dot(h, w2_ref[k], preferred_element_type=jnp.float32, precision=lax.Precision.DEFAULT)
            acc = acc + jnp.dot(ptl_ref[k], res, preferred_element_type=jnp.float32, precision=lax.Precision.DEFAULT)
        out_ref[...] = acc

        rdma_r = []
        for k in range(EXPERTS_PER_SHARD):
            rdma_x[k].wait_send()
            rdma_x[k].wait_recv()
            h = jnp.maximum(jnp.dot(xin[k], w1_ref[k], preferred_element_type=jnp.float32, precision=lax.Precision.DEFAULT), 0.0)
            resout[k] = jnp.dot(h, w2_ref[k], preferred_element_type=jnp.float32, precision=lax.Precision.DEFAULT)
            r = pltpu.make_async_remote_copy(
                src_ref=resout.at[k], dst_ref=resb.at[k],
                send_sem=send_sems.at[EXPERTS_PER_SHARD + k],
                recv_sem=recv_sems.at[EXPERTS_PER_SHARD + k],
                device_id=peer, device_id_type=pl.DeviceIdType.MESH)
            r.start()
            rdma_r.append(r)

        for k in range(EXPERTS_PER_SHARD):
            rdma_r[k].wait_recv()
            out_ref[...] = out_ref[...] + jnp.dot(
                pto_ref[k], resb[k], preferred_element_type=jnp.float32, precision=lax.Precision.DEFAULT)
        for r in rdma_r:
            r.wait_send()

    return pl.pallas_call(
        body,
        out_shape=jax.ShapeDtypeStruct((tokens, d_model), x.dtype),
        in_specs=[pl.BlockSpec(memory_space=pltpu.VMEM)] * 7,
        out_specs=pl.BlockSpec(memory_space=pltpu.VMEM),
        scratch_shapes=[
            pltpu.VMEM((EXPERTS_PER_SHARD, CAP, d_model), x.dtype),
            pltpu.VMEM((EXPERTS_PER_SHARD, CAP, d_model), x.dtype),
            pltpu.VMEM((EXPERTS_PER_SHARD, CAP, d_model), x.dtype),
            pltpu.SemaphoreType.DMA((2 * EXPERTS_PER_SHARD,)),
            pltpu.SemaphoreType.DMA((2 * EXPERTS_PER_SHARD,)),
        ],
        compiler_params=pltpu.CompilerParams(
            collective_id=0,
            vmem_limit_bytes=100 * 1024 * 1024,
        ),
    )(x, p_loc, p_out, pt_loc, pt_out, W1, W2)


# device time: 74647 ns/iter; 6.8934x vs baseline; 1.2702x over previous
import jax
import jax.numpy as jnp
from jax import lax
from jax.experimental import pallas as pl
from jax.experimental.pallas import tpu as pltpu

N_EXPERTS = 4
EXPERTS_PER_SHARD = 2
CAP = 320


def kernel(x, assign, W1, W2):
    tokens, d_model = x.shape
    my_x = lax.axis_index("x")

    oh = (assign[:, None] == jnp.arange(N_EXPERTS, dtype=assign.dtype)[None, :])
    pos = jnp.cumsum(oh.astype(jnp.int32), axis=0) - 1
    sel = oh.T[:, None, :] & (pos.T[:, None, :] == jnp.arange(CAP, dtype=jnp.int32)[None, :, None])
    P = sel.astype(jnp.bfloat16)
    PT = jnp.swapaxes(P, 1, 2)

    loc0 = EXPERTS_PER_SHARD * my_x
    out0 = EXPERTS_PER_SHARD * (1 - my_x)
    p_loc = lax.dynamic_slice(P, (loc0, 0, 0), (EXPERTS_PER_SHARD, CAP, tokens))
    p_out = lax.dynamic_slice(P, (out0, 0, 0), (EXPERTS_PER_SHARD, CAP, tokens))
    pt_loc = lax.dynamic_slice(PT, (loc0, 0, 0), (EXPERTS_PER_SHARD, tokens, CAP))
    pt_out = lax.dynamic_slice(PT, (out0, 0, 0), (EXPERTS_PER_SHARD, tokens, CAP))

    f32 = jnp.float32
    bf16 = jnp.bfloat16

    def body(x_ref, pl_ref, po_ref, ptl_ref, pto_ref, w1_ref, w2_ref,
             out_ref, xin, resout, resb, send_sems, recv_sems):
        mx = lax.axis_index("x")
        my = lax.axis_index("y")
        mz = lax.axis_index("z")
        peer = (1 - mx, my, mz)

        barrier_sem = pltpu.get_barrier_semaphore()
        pl.semaphore_signal(barrier_sem, inc=1, device_id=peer,
                            device_id_type=pl.DeviceIdType.MESH)
        pl.semaphore_wait(barrier_sem, 1)

        xb = x_ref[...].astype(bf16)

        rdma_x = []
        for k in range(EXPERTS_PER_SHARD):
            resout[k] = jnp.dot(po_ref[k], xb, preferred_element_type=f32).astype(bf16)
            r = pltpu.make_async_remote_copy(
                src_ref=resout.at[k], dst_ref=xin.at[k],
                send_sem=send_sems.at[k], recv_sem=recv_sems.at[k],
                device_id=peer, device_id_type=pl.DeviceIdType.MESH)
            r.start()
            rdma_x.append(r)

        acc = jnp.zeros((tokens, d_model), f32)
        for k in range(EXPERTS_PER_SHARD):
            xg = jnp.dot(pl_ref[k], xb, preferred_element_type=f32).astype(bf16)
            h = jnp.maximum(jnp.dot(xg, w1_ref[k], preferred_element_type=f32), 0.0)
            res = jnp.dot(h.astype(bf16), w2_ref[k], preferred_element_type=f32)
            acc = acc + jnp.dot(ptl_ref[k], res.astype(bf16), preferred_element_type=f32)
        out_ref[...] = acc

        rdma_r = []
        for k in range(EXPERTS_PER_SHARD):
            rdma_x[k].wait_send()
            rdma_x[k].wait_recv()
            h = jnp.maximum(jnp.dot(xin[k], w1_ref[k], preferred_element_type=f32), 0.0)
            resout[k] = jnp.dot(h.astype(bf16), w2_ref[k], preferred_element_type=f32).astype(bf16)
            r = pltpu.make_async_remote_copy(
                src_ref=resout.at[k], dst_ref=resb.at[k],
                send_sem=send_sems.at[EXPERTS_PER_SHARD + k],
                recv_sem=recv_sems.at[EXPERTS_PER_SHARD + k],
                device_id=peer, device_id_type=pl.DeviceIdType.MESH)
            r.start()
            rdma_r.append(r)

        for k in range(EXPERTS_PER_SHARD):
            rdma_r[k].wait_recv()
            out_ref[...] = out_ref[...] + jnp.dot(
                pto_ref[k], resb[k], preferred_element_type=f32)
        for r in rdma_r:
            r.wait_send()

    return pl.pallas_call(
        body,
        out_shape=jax.ShapeDtypeStruct((tokens, d_model), x.dtype),
        in_specs=[pl.BlockSpec(memory_space=pltpu.VMEM)] * 7,
        out_specs=pl.BlockSpec(memory_space=pltpu.VMEM),
        scratch_shapes=[
            pltpu.VMEM((EXPERTS_PER_SHARD, CAP, d_model), bf16),
            pltpu.VMEM((EXPERTS_PER_SHARD, CAP, d_model), bf16),
            pltpu.VMEM((EXPERTS_PER_SHARD, CAP, d_model), bf16),
            pltpu.SemaphoreType.DMA((2 * EXPERTS_PER_SHARD,)),
            pltpu.SemaphoreType.DMA((2 * EXPERTS_PER_SHARD,)),
        ],
        compiler_params=pltpu.CompilerParams(
            collective_id=0,
            vmem_limit_bytes=60 * 1024 * 1024,
        ),
    )(x, p_loc, p_out, pt_loc, pt_out,
      W1.astype(jnp.bfloat16), W2.astype(jnp.bfloat16))


# device time: 57495 ns/iter; 8.9499x vs baseline; 1.2983x over previous
import jax
import jax.numpy as jnp
from jax import lax
from jax.experimental import pallas as pl
from jax.experimental.pallas import tpu as pltpu

N_EXPERTS = 4
EXPERTS_PER_SHARD = 2
CAP = 320


def kernel(x, assign, W1, W2):
    tokens, d_model = x.shape
    my_x = lax.axis_index("x")

    oh = (assign[:, None] == jnp.arange(N_EXPERTS, dtype=assign.dtype)[None, :]).astype(jnp.int32)
    pos = ((jnp.cumsum(oh, axis=0) - 1) * oh).sum(axis=1)
    ap_row = jnp.stack([assign.astype(jnp.int32), pos])
    ap_col = ap_row.T

    f32 = jnp.float32
    bf16 = jnp.bfloat16

    def body(x_ref, apr_ref, apc_ref, w1_ref, w2_ref,
             out_ref, xin, resout, resb, send_sems, recv_sems):
        mx = lax.axis_index("x")
        my = lax.axis_index("y")
        mz = lax.axis_index("z")
        peer = (1 - mx, my, mz)

        def make_p(e):
            iota = lax.broadcasted_iota(jnp.int32, (CAP, tokens), 0)
            sel = (apr_ref[0:1, :] == e) & (apr_ref[1:2, :] == iota)
            return sel.astype(bf16)

        def make_pt(e):
            iota = lax.broadcasted_iota(jnp.int32, (tokens, CAP), 1)
            sel = (apc_ref[:, 0:1] == e) & (apc_ref[:, 1:2] == iota)
            return sel.astype(bf16)

        barrier_sem = pltpu.get_barrier_semaphore()
        pl.semaphore_signal(barrier_sem, inc=1, device_id=peer,
                            device_id_type=pl.DeviceIdType.MESH)
        pl.semaphore_wait(barrier_sem, 1)

        xb = x_ref[...].astype(bf16)

        rdma_x = []
        for k in range(EXPERTS_PER_SHARD):
            e_out = 2 * (1 - mx) + k
            resout[k] = jnp.dot(make_p(e_out), xb, preferred_element_type=f32).astype(bf16)
            r = pltpu.make_async_remote_copy(
                src_ref=resout.at[k], dst_ref=xin.at[k],
                send_sem=send_sems.at[k], recv_sem=recv_sems.at[k],
                device_id=peer, device_id_type=pl.DeviceIdType.MESH)
            r.start()
            rdma_x.append(r)

        acc = jnp.zeros((tokens, d_model), f32)
        for k in range(EXPERTS_PER_SHARD):
            e_loc = 2 * mx + k
            xg = jnp.dot(make_p(e_loc), xb, preferred_element_type=f32)
            h = jnp.maximum(jnp.dot(xg, w1_ref[k], preferred_element_type=f32), 0.0)
            res = jnp.dot(h, w2_ref[k], preferred_element_type=f32)
            acc = acc + jnp.dot(make_pt(e_loc), res.astype(bf16), preferred_element_type=f32)
        out_ref[...] = acc

        rdma_r = []
        for k in range(EXPERTS_PER_SHARD):
            rdma_x[k].wait_send()
            rdma_x[k].wait_recv()
            xp = xin[k][...].astype(f32)
            h = jnp.maximum(jnp.dot(xp, w1_ref[k], preferred_element_type=f32), 0.0)
            resout[k] = jnp.dot(h, w2_ref[k], preferred_element_type=f32).astype(bf16)
            r = pltpu.make_async_remote_copy(
                src_ref=resout.at[k], dst_ref=resb.at[k],
                send_sem=send_sems.at[EXPERTS_PER_SHARD + k],
                recv_sem=recv_sems.at[EXPERTS_PER_SHARD + k],
                device_id=peer, device_id_type=pl.DeviceIdType.MESH)
            r.start()
            rdma_r.append(r)

        for k in range(EXPERTS_PER_SHARD):
            e_out = 2 * (1 - mx) + k
            rdma_r[k].wait_recv()
            out_ref[...] = out_ref[...] + jnp.dot(
                make_pt(e_out), resb[k], preferred_element_type=f32)
        for r in rdma_r:
            r.wait_send()

    return pl.pallas_call(
        body,
        out_shape=jax.ShapeDtypeStruct((tokens, d_model), x.dtype),
        in_specs=[pl.BlockSpec(memory_space=pltpu.VMEM)] * 5,
        out_specs=pl.BlockSpec(memory_space=pltpu.VMEM),
        scratch_shapes=[
            pltpu.VMEM((EXPERTS_PER_SHARD, CAP, d_model), bf16),
            pltpu.VMEM((EXPERTS_PER_SHARD, CAP, d_model), bf16),
            pltpu.VMEM((EXPERTS_PER_SHARD, CAP, d_model), bf16),
            pltpu.SemaphoreType.DMA((2 * EXPERTS_PER_SHARD,)),
            pltpu.SemaphoreType.DMA((2 * EXPERTS_PER_SHARD,)),
        ],
        compiler_params=pltpu.CompilerParams(
            collective_id=0,
            vmem_limit_bytes=60 * 1024 * 1024,
        ),
    )(x, ap_row, ap_col, W1, W2)
